# baseline (speedup 1.0000x reference)
import functools
import numpy as np
import jax
import jax.numpy as jnp
from jax import lax
from jax.experimental import pallas as pl
from jax.experimental.pallas import tpu as pltpu
from jax.experimental.pallas import tpu_sc as plsc

EPS = 1e-5
NW = 32
CHUNKS = ((0, 120), (120, 80))
CMAX = 120
LPAD = 216


def _make_pe(max_len, d):
    position = np.arange(max_len, dtype=np.float32)[:, None]
    div_term = np.exp(np.arange(0, d, 2, dtype=np.float32) * -(np.log(10000.0) / d))
    pe = np.zeros((max_len, d), dtype=np.float32)
    pe[:, 0::2] = np.sin(position * div_term)
    pe[:, 1::2] = np.cos(position * div_term)
    return pe


def _build(B, L, D):
    assert B % NW == 0 and D % 16 == 0
    assert sum(sz for _, sz in CHUNKS) == L
    N = B // NW
    assert N % 2 == 0
    nk = D // 16

    @functools.partial(
        pl.kernel,
        mesh=plsc.VectorSubcoreMesh(core_axis_name="c", subcore_axis_name="s"),
        out_type=jax.ShapeDtypeStruct((B, L, D), jnp.float32),
        compiler_params=pltpu.CompilerParams(use_tc_tiling_on_sc=False),
        scratch_types=[
            pltpu.VMEM((2, LPAD), jnp.int32),
            pltpu.VMEM((2, LPAD), jnp.int32),
            pltpu.VMEM((2, LPAD), jnp.int32),
            pltpu.VMEM((2, LPAD), jnp.int32),
            pltpu.VMEM((2, CMAX, 2 * D), jnp.float32),
            pltpu.VMEM((2, CMAX, 2 * D), jnp.float32),
            pltpu.VMEM((L, D), jnp.float32),
            pltpu.VMEM((2, CMAX, D), jnp.float32),
            pltpu.VMEM((D,), jnp.float32),
            pltpu.VMEM((D,), jnp.float32),
            pltpu.SemaphoreType.DMA,
            pltpu.SemaphoreType.DMA,
            pltpu.SemaphoreType.DMA,
            pltpu.SemaphoreType.DMA,
            pltpu.SemaphoreType.DMA,
            pltpu.SemaphoreType.DMA,
        ],
    )
    def _k(xs_hbm, ts_hbm, xh_hbm, th_hbm, tok_hbm, tim_hbm, pe_hbm,
           g_hbm, b_hbm, out_hbm,
           xsb, tsb, xhb, thb, tokb, timb, peb, outb, gb, bb,
           si0, si1, sg0, sg1, so0, so1):
        si = [si0, si1]
        sg = [sg0, sg1]
        so = [so0, so1]
        wid = lax.axis_index("s") * 2 + lax.axis_index("c")
        pltpu.sync_copy(pe_hbm, peb)
        pltpu.sync_copy(g_hbm, gb)
        pltpu.sync_copy(b_hbm, bb)
        gvs = [gb[pl.ds(16 * k, 16)] for k in range(nk)]
        bvs = [bb[pl.ds(16 * k, 16)] for k in range(nk)]
        base = wid * N
        lane = lax.broadcasted_iota(jnp.int32, (16,), 0)
        perms = [(lane + sh) & 15 for sh in (8, 4, 2, 1)]
        zperm = lane * 0
        dnums = lax.GatherDimensionNumbers(
            offset_dims=(), collapsed_slice_dims=(0,), start_index_map=(0,))

        def shuffle(v, p):
            return lax.gather(v, p[:, None], dnums, (1,),
                              mode=lax.GatherScatterMode.PROMISE_IN_BOUNDS)

        def lanesum(v):
            for p in perms:
                v = v + shuffle(v, p)
            return v

        def issue_idx(p, seq):
            dst = pl.ds(0, L)
            pltpu.async_copy(xs_hbm.at[seq], xsb.at[p, dst], si[p])
            pltpu.async_copy(ts_hbm.at[seq], tsb.at[p, dst], si[p])
            pltpu.async_copy(xh_hbm.at[seq], xhb.at[p, dst], si[p])
            pltpu.async_copy(th_hbm.at[seq], thb.at[p, dst], si[p])

        def wait_idx(p):
            dst = pl.ds(0, L)
            for ref in (xsb, tsb, xhb, thb):
                pltpu.make_async_copy(xs_hbm.at[0], ref.at[p, dst], si[p]).wait()

        def issue_gather(p, c):
            off, sz = CHUNKS[c]
            sl = pl.ds(off, sz)
            dst = pl.ds(0, sz)
            pltpu.async_copy(tok_hbm.at[xsb.at[p, sl]], tokb.at[c, dst], sg[c])
            pltpu.async_copy(tim_hbm.at[tsb.at[p, sl]], timb.at[c, dst], sg[c])

        def wait_gather(c):
            _, sz = CHUNKS[c]
            dst = pl.ds(0, sz)
            pltpu.make_async_copy(tok_hbm.at[pl.ds(0, sz)], tokb.at[c, dst], sg[c]).wait()
            pltpu.make_async_copy(tim_hbm.at[pl.ds(0, sz)], timb.at[c, dst], sg[c]).wait()

        def wait_out(c):
            off, sz = CHUNKS[c]
            pltpu.make_async_copy(outb.at[c, pl.ds(0, sz)],
                                  out_hbm.at[0, pl.ds(off, sz)], so[c]).wait()

        def compute(p, c, seq):
            off, sz = CHUNKS[c]

            def row_body(r, carry):
                hx = shuffle(xhb[p, pl.ds(off + r, 16)], zperm).astype(jnp.float32)
                ht = shuffle(thb[p, pl.ds(off + r, 16)], zperm).astype(jnp.float32)
                e = []
                for k in range(nk):
                    lo = pl.ds(16 * k, 16)
                    hi = pl.ds(D + 16 * k, 16)
                    tklo = tokb[c, r, lo]
                    tmlo = timb[c, r, lo]
                    tk = tklo + (tokb[c, r, hi] - tklo) * hx
                    tm = tmlo + (timb[c, r, hi] - tmlo) * ht
                    e.append(tk + tm + peb[off + r, lo])
                s = (e[0] + e[1]) + (e[2] + e[3])
                q = (e[0] * e[0] + e[1] * e[1]) + (e[2] * e[2] + e[3] * e[3])
                inv_d = jnp.float32(1.0 / D)
                mu = lanesum(s) * inv_d
                ms = lanesum(q) * inv_d
                var = ms - mu * mu
                xx = var + jnp.float32(EPS)
                i = lax.bitcast_convert_type(xx, jnp.int32)
                i = jnp.int32(0x5F3759DF) - lax.shift_right_arithmetic(i, 1)
                y = lax.bitcast_convert_type(i, jnp.float32)
                for _ in range(3):
                    y = y * (jnp.float32(1.5) - jnp.float32(0.5) * xx * y * y)
                for k in range(nk):
                    sl = pl.ds(16 * k, 16)
                    outb[c, r, sl] = (e[k] - mu) * y * gvs[k] + bvs[k]
                return carry

            lax.fori_loop(0, sz, row_body, 0)
            pltpu.async_copy(outb.at[c, pl.ds(0, sz)],
                             out_hbm.at[seq, pl.ds(off, sz)], so[c])

        issue_idx(0, base)
        issue_idx(1, base + 1)
        wait_idx(0)
        issue_gather(0, 0)

        def seq_step(sp, i2, s):
            wait_gather(0)
            issue_gather(sp, 1)
            @pl.when(jnp.logical_not(jnp.logical_and(i2 == 0, sp == 0)))
            def _():
                wait_out(0)
            compute(sp, 0, s)
            wait_gather(1)
            if sp == 0:
                wait_idx(1)
                issue_gather(1, 0)
            else:
                @pl.when(i2 < N // 2 - 1)
                def _():
                    wait_idx(0)
                    issue_gather(0, 0)
            @pl.when(jnp.logical_not(jnp.logical_and(i2 == 0, sp == 0)))
            def _():
                wait_out(1)
            compute(sp, 1, s)
            @pl.when(s + 2 < base + N)
            def _():
                issue_idx(sp, s + 2)

        def step(i2, carry):
            seq_step(0, i2, base + 2 * i2)
            seq_step(1, i2, base + 2 * i2 + 1)
            return carry

        lax.fori_loop(0, N // 2, step, 0)
        wait_out(0)
        wait_out(1)

    return _k


def kernel(x, timestamp, tok_table, time_table, gamma, beta):
    B, L = x.shape
    D = tok_table.shape[1]
    V, T = tok_table.shape[0], time_table.shape[0]
    pe = jnp.asarray(_make_pe(L, D))
    tok2 = tok_table.reshape(V // 2, 2 * D)
    tim2 = time_table.reshape(T // 2, 2 * D)
    return _build(B, L, D)(
        x >> 1, timestamp >> 1, x & 1, timestamp & 1,
        tok2, tim2, pe, gamma, beta)


# speedup vs baseline: 1.7121x; 1.7121x over previous
import functools
import numpy as np
import jax
import jax.numpy as jnp
from jax import lax
from jax.experimental import pallas as pl
from jax.experimental.pallas import tpu as pltpu
from jax.experimental.pallas import tpu_sc as plsc

EPS = 1e-5
NW = 32
CHUNKS = ((0, 120), (120, 80))


def _make_pe(max_len, d):
    position = np.arange(max_len, dtype=np.float32)[:, None]
    div_term = np.exp(np.arange(0, d, 2, dtype=np.float32) * -(np.log(10000.0) / d))
    pe = np.zeros((max_len, d), dtype=np.float32)
    pe[:, 0::2] = np.sin(position * div_term)
    pe[:, 1::2] = np.cos(position * div_term)
    return pe


def _build(B, L, D):
    assert B % NW == 0 and D % 16 == 0
    assert sum(sz for _, sz in CHUNKS) == L
    N = B // NW
    assert N % 2 == 0
    nk = D // 16

    @functools.partial(
        pl.kernel,
        mesh=plsc.VectorSubcoreMesh(core_axis_name="c", subcore_axis_name="s"),
        out_type=jax.ShapeDtypeStruct((B, L, D), jnp.float32),
        compiler_params=pltpu.CompilerParams(use_tc_tiling_on_sc=False),
        scratch_types=[
            pltpu.VMEM((2, L), jnp.int32),
            pltpu.VMEM((2, L), jnp.int32),
            pltpu.VMEM((2, L, D), jnp.float32),
            pltpu.VMEM((2, L, D), jnp.float32),
            pltpu.VMEM((L, D), jnp.float32),
            pltpu.VMEM((2, L, D), jnp.float32),
            pltpu.VMEM((D,), jnp.float32),
            pltpu.VMEM((D,), jnp.float32),
            pltpu.SemaphoreType.DMA,
            pltpu.SemaphoreType.DMA,
            pltpu.SemaphoreType.DMA,
            pltpu.SemaphoreType.DMA,
            pltpu.SemaphoreType.DMA,
            pltpu.SemaphoreType.DMA,
        ],
    )
    def _k(x_hbm, ts_hbm, tok_hbm, tim_hbm, pe_hbm, g_hbm, b_hbm, out_hbm,
           xidx, tidx, tokb, timb, peb, outb, gb, bb,
           si0, si1, sg0, sg1, so0, so1):
        si = [si0, si1]
        sg = [sg0, sg1]
        so = [so0, so1]
        wid = lax.axis_index("s") * 2 + lax.axis_index("c")
        pltpu.sync_copy(pe_hbm, peb)
        pltpu.sync_copy(g_hbm, gb)
        pltpu.sync_copy(b_hbm, bb)
        gvs = [gb[pl.ds(16 * k, 16)] for k in range(nk)]
        bvs = [bb[pl.ds(16 * k, 16)] for k in range(nk)]
        base = wid * N
        lane = lax.broadcasted_iota(jnp.int32, (16,), 0)
        perms = [(lane + sh) & 15 for sh in (8, 4, 2, 1)]
        dnums = lax.GatherDimensionNumbers(
            offset_dims=(), collapsed_slice_dims=(0,), start_index_map=(0,))

        def shuffle(v, p):
            return lax.gather(v, p[:, None], dnums, (1,),
                              mode=lax.GatherScatterMode.PROMISE_IN_BOUNDS)

        def lanesum(v):
            for p in perms:
                v = v + shuffle(v, p)
            return v

        def issue_idx(b, seq):
            pltpu.async_copy(x_hbm.at[seq], xidx.at[b], si[b])
            pltpu.async_copy(ts_hbm.at[seq], tidx.at[b], si[b])

        def wait_idx(b):
            pltpu.make_async_copy(x_hbm.at[0], xidx.at[b], si[b]).wait()
            pltpu.make_async_copy(ts_hbm.at[0], tidx.at[b], si[b]).wait()

        def issue_gather(b):
            for off, sz in CHUNKS:
                dst = pl.ds(off, sz)
                pltpu.async_copy(tok_hbm.at[xidx.at[b, dst]], tokb.at[b, dst], sg[b])
                pltpu.async_copy(tim_hbm.at[tidx.at[b, dst]], timb.at[b, dst], sg[b])

        def wait_gather(b):
            for off, sz in CHUNKS:
                dst = pl.ds(off, sz)
                pltpu.make_async_copy(tok_hbm.at[pl.ds(0, sz)], tokb.at[b, dst], sg[b]).wait()
                pltpu.make_async_copy(tim_hbm.at[pl.ds(0, sz)], timb.at[b, dst], sg[b]).wait()

        def wait_out(b):
            pltpu.make_async_copy(outb.at[b], out_hbm.at[0], so[b]).wait()

        def compute(b, seq):
            def row_body(r, c2):
                e = []
                for k in range(nk):
                    sl = pl.ds(16 * k, 16)
                    e.append(tokb[b, r, sl] + timb[b, r, sl] + peb[r, sl])
                s = (e[0] + e[1]) + (e[2] + e[3])
                q = (e[0] * e[0] + e[1] * e[1]) + (e[2] * e[2] + e[3] * e[3])
                inv_d = jnp.float32(1.0 / D)
                mu = lanesum(s) * inv_d
                ms = lanesum(q) * inv_d
                var = ms - mu * mu
                xx = var + jnp.float32(EPS)
                i = lax.bitcast_convert_type(xx, jnp.int32)
                i = jnp.int32(0x5F3759DF) - lax.shift_right_arithmetic(i, 1)
                y = lax.bitcast_convert_type(i, jnp.float32)
                for _ in range(3):
                    y = y * (jnp.float32(1.5) - jnp.float32(0.5) * xx * y * y)
                for k in range(nk):
                    sl = pl.ds(16 * k, 16)
                    outb[b, r, sl] = (e[k] - mu) * y * gvs[k] + bvs[k]
                return c2

            lax.fori_loop(0, L, row_body, 0)
            pltpu.async_copy(outb.at[b], out_hbm.at[seq], so[b])

        issue_idx(0, base)
        issue_idx(1, base + 1)
        wait_idx(0)
        issue_gather(0)

        def step(i, carry):
            for b in range(2):
                t = 2 * i + b
                last = (i >= N // 2 - 1)
                if b == 0:
                    wait_idx(1)
                    issue_gather(1)
                else:
                    @pl.when(jnp.logical_not(last))
                    def _():
                        wait_idx(0)
                        issue_gather(0)
                wait_gather(b)
                @pl.when(jnp.logical_not(last))
                def _():
                    issue_idx(b, base + t + 2)
                @pl.when(i >= 1)
                def _():
                    wait_out(b)
                compute(b, base + t)
            return carry

        lax.fori_loop(0, N // 2, step, 0)
        wait_out(0)
        wait_out(1)

    return _k


def kernel(x, timestamp, tok_table, time_table, gamma, beta):
    B, L = x.shape
    D = tok_table.shape[1]
    pe = jnp.asarray(_make_pe(L, D))
    return _build(B, L, D)(x, timestamp, tok_table, time_table, pe, gamma, beta)
